# baseline (speedup 1.0000x reference)
import functools

import jax
import jax.numpy as jnp
from jax import lax
from jax.experimental import pallas as pl
from jax.experimental.pallas import tpu as pltpu
from jax.experimental.pallas import tpu_sc as plsc

_N_ROWS = 100000
_EMB = 64
_NC = 2
_NS = 16
_NW = _NC * _NS
_ROWS_PER_W = 3200
_CHUNK = 400
_NCHUNK = _ROWS_PER_W // _CHUNK


def _sc_copy(w_hbm, out_hbm, buf0, buf1, in0, in1, out0, out1):
    cid = lax.axis_index("c")
    sid = lax.axis_index("s")
    wid = sid * _NC + cid
    base = jnp.minimum(wid * _ROWS_PER_W, _N_ROWS - _ROWS_PER_W)

    bufs = (buf0, buf1)
    isems = (in0, in1)
    osems = (out0, out1)

    def in_copy(k, b):
        return pltpu.make_async_copy(
            w_hbm.at[pl.ds(base + k * _CHUNK, _CHUNK), :], bufs[b], isems[b])

    def out_copy(k, b):
        return pltpu.make_async_copy(
            bufs[b], out_hbm.at[pl.ds(base + k * _CHUNK, _CHUNK), :], osems[b])

    in_copy(0, 0).start()
    for k in range(_NCHUNK):
        b = k % 2
        nb = (k + 1) % 2
        if k + 1 < _NCHUNK:
            if k + 1 >= 2:
                out_copy(k - 1, nb).wait()
            in_copy(k + 1, nb).start()
        in_copy(k, b).wait()
        out_copy(k, b).start()
    out_copy(_NCHUNK - 2, (_NCHUNK - 2) % 2).wait()
    out_copy(_NCHUNK - 1, (_NCHUNK - 1) % 2).wait()


def kernel(weight):
    n, d = weight.shape
    run = pl.kernel(
        _sc_copy,
        out_type=jax.ShapeDtypeStruct((n, d), weight.dtype),
        mesh=plsc.VectorSubcoreMesh(
            core_axis_name="c", subcore_axis_name="s",
            num_cores=_NC, num_subcores=_NS),
        scratch_types=[
            pltpu.VMEM((_CHUNK, _EMB), jnp.float32),
            pltpu.VMEM((_CHUNK, _EMB), jnp.float32),
            pltpu.SemaphoreType.DMA,
            pltpu.SemaphoreType.DMA,
            pltpu.SemaphoreType.DMA,
            pltpu.SemaphoreType.DMA,
        ],
    )
    return run(weight)
